# baseline (device time: 24178 ns/iter reference)
import jax
import jax.numpy as jnp
from jax import lax
from jax.experimental import pallas as pl
from jax.experimental.pallas import tpu as pltpu

UNROLL = 8


def kernel(ids, E):
    T = ids.shape[0]
    V_LOC, D = E.shape

    def body(ids_smem, ids_vmem, e_hbm, out_ref,
             gbuf, send_buf, recv_buf, gather_sem, send_sem, recv_sem):
        my_x = lax.axis_index("x")
        my_y = lax.axis_index("y")
        my_z = lax.axis_index("z")
        partner = (1 - my_x, my_y, my_z)

        barrier_sem = pltpu.get_barrier_semaphore()
        pl.semaphore_signal(
            barrier_sem, inc=1,
            device_id=partner, device_id_type=pl.DeviceIdType.MESH,
        )
        pl.semaphore_wait(barrier_sem, 1)

        off = my_x * V_LOC

        def issue(r, carry):
            for j in range(UNROLL):
                i = r * UNROLL + j
                idx = jnp.clip(ids_smem[i] - off, 0, V_LOC - 1)
                pltpu.make_async_copy(
                    e_hbm.at[pl.ds(idx, 1), :],
                    gbuf.at[pl.ds(i, 1), :],
                    gather_sem,
                ).start()
            return carry
        lax.fori_loop(0, T // UNROLL, issue, 0)

        def drain(i, carry):
            pltpu.make_async_copy(
                e_hbm.at[pl.ds(0, 1), :], gbuf.at[pl.ds(0, 1), :], gather_sem
            ).wait()
            return carry
        lax.fori_loop(0, T, drain, 0)

        ids_col = ids_vmem[:].reshape(T, 1)
        mask = ((ids_col >= off) & (ids_col < off + V_LOC)).astype(jnp.float32)
        part = gbuf[:, :] * mask
        send_buf[:, :] = part.astype(jnp.bfloat16)

        rdma = pltpu.make_async_remote_copy(
            src_ref=send_buf,
            dst_ref=recv_buf,
            send_sem=send_sem,
            recv_sem=recv_sem,
            device_id=partner,
            device_id_type=pl.DeviceIdType.MESH,
        )
        rdma.start()
        rdma.wait()

        out_ref[:, :] = part + recv_buf[:, :].astype(jnp.float32)

    return pl.pallas_call(
        body,
        out_shape=jax.ShapeDtypeStruct((T, D), jnp.float32),
        in_specs=[
            pl.BlockSpec(memory_space=pltpu.SMEM),
            pl.BlockSpec(memory_space=pltpu.VMEM),
            pl.BlockSpec(memory_space=pl.ANY),
        ],
        out_specs=pl.BlockSpec(memory_space=pltpu.VMEM),
        scratch_shapes=[
            pltpu.VMEM((T, D), jnp.float32),
            pltpu.VMEM((T, D), jnp.bfloat16),
            pltpu.VMEM((T, D), jnp.bfloat16),
            pltpu.SemaphoreType.DMA,
            pltpu.SemaphoreType.DMA,
            pltpu.SemaphoreType.DMA,
        ],
        compiler_params=pltpu.CompilerParams(collective_id=0),
    )(ids, ids, E)


# device time: 15171 ns/iter; 1.5937x vs baseline; 1.5937x over previous
import jax
import jax.numpy as jnp
from jax import lax
from jax.experimental import pallas as pl
from jax.experimental.pallas import tpu as pltpu

N_CHUNKS = 4
VOCAB_SLAB = 1024


def kernel(ids, E):
    T = ids.shape[0]
    V_LOC, D = E.shape
    TC = T // N_CHUNKS

    E_bf = E.astype(jnp.bfloat16)

    def body(ids_ref, e_ref, out_ref,
             send_buf, recv_buf, send_sems, recv_sems):
        my_x = lax.axis_index("x")
        my_y = lax.axis_index("y")
        my_z = lax.axis_index("z")
        partner = (1 - my_x, my_y, my_z)

        barrier_sem = pltpu.get_barrier_semaphore()
        pl.semaphore_signal(
            barrier_sem, inc=1,
            device_id=partner, device_id_type=pl.DeviceIdType.MESH,
        )
        pl.semaphore_wait(barrier_sem, 1)

        ids_col = ids_ref[:].reshape(T, 1) - my_x * V_LOC

        accs = []
        rdmas = []
        for c in range(N_CHUNKS):
            rows = slice(c * TC, (c + 1) * TC)
            idc = ids_col[rows]
            acc = None
            for k in range(V_LOC // VOCAB_SLAB):
                iota = lax.broadcasted_iota(jnp.int32, (TC, VOCAB_SLAB), 1)
                oh = (iota + k * VOCAB_SLAB == idc).astype(jnp.bfloat16)
                part = lax.dot_general(
                    oh, e_ref[k * VOCAB_SLAB:(k + 1) * VOCAB_SLAB, :],
                    (((1,), (0,)), ((), ())),
                    preferred_element_type=jnp.float32,
                )
                acc = part if acc is None else acc + part
            accs.append(acc)
            send_buf[c] = acc.astype(jnp.bfloat16)
            rdma = pltpu.make_async_remote_copy(
                src_ref=send_buf.at[c],
                dst_ref=recv_buf.at[c],
                send_sem=send_sems.at[c],
                recv_sem=recv_sems.at[c],
                device_id=partner,
                device_id_type=pl.DeviceIdType.MESH,
            )
            rdma.start()
            rdmas.append(rdma)

        for c in range(N_CHUNKS):
            rows = slice(c * TC, (c + 1) * TC)
            rdmas[c].wait_recv()
            out_ref[rows, :] = accs[c] + recv_buf[c].astype(jnp.float32)
        for c in range(N_CHUNKS):
            rdmas[c].wait_send()

    return pl.pallas_call(
        body,
        out_shape=jax.ShapeDtypeStruct((T, D), jnp.float32),
        in_specs=[
            pl.BlockSpec(memory_space=pltpu.VMEM),
            pl.BlockSpec(memory_space=pltpu.VMEM),
        ],
        out_specs=pl.BlockSpec(memory_space=pltpu.VMEM),
        scratch_shapes=[
            pltpu.VMEM((N_CHUNKS, TC, D), jnp.bfloat16),
            pltpu.VMEM((N_CHUNKS, TC, D), jnp.bfloat16),
            pltpu.SemaphoreType.DMA((N_CHUNKS,)),
            pltpu.SemaphoreType.DMA((N_CHUNKS,)),
        ],
        compiler_params=pltpu.CompilerParams(collective_id=0),
    )(ids, E_bf)
